# Initial kernel scaffold; baseline (speedup 1.0000x reference)
import jax
import jax.numpy as jnp
from jax.experimental import pallas as pl


def kernel(query_x, query_e, graph_x, matches, inq_W, inq_b, gin_W, gin_b, lin_W, lin_b, out_W, out_b):
    raise NotImplementedError("write your pallas kernel here")


# speedup vs baseline: 2.8906x; 2.8906x over previous
import dataclasses
import functools
import jax
import jax.numpy as jnp
from jax import lax
from jax.experimental import pallas as pl
from jax.experimental.pallas import tpu as pltpu
from jax.experimental.pallas import tpu_sc as plsc

NQ = 10000
E = 160000
NG = 50000
D = 256
H = 128
M = 5000

NC = 2
NS = 16
LANES = 16

ECH = 128
N_ECH = E // ECH
RCH = 80
N_RCH = NQ // RCH

RB = 1000
GRID = NQ // RB

_mesh = plsc.VectorSubcoreMesh(core_axis_name="c", subcore_axis_name="s")

_no_layout_cp = pltpu.CompilerParams()
if "needs_layout_passes" in pltpu.CompilerParams.__dataclass_fields__:
    _no_layout_cp = dataclasses.replace(_no_layout_cp, needs_layout_passes=False)


MPAD = ((M + LANES - 1) // LANES) * LANES
N_MCH = MPAD // LANES


@functools.partial(
    pl.kernel,
    out_type=jax.ShapeDtypeStruct((NQ,), jnp.int32),
    mesh=_mesh,
    compiler_params=_no_layout_cp,
    scratch_types=[
        pltpu.VMEM((NQ,), jnp.int32),
        pltpu.VMEM((MPAD,), jnp.int32),
        pltpu.VMEM((MPAD,), jnp.int32),
    ],
)
def _build_idx(m0_hbm, m1_hbm, idx_hbm, idx_v, m0_v, m1_v):
    c = lax.axis_index("c")
    s = lax.axis_index("s")

    @pl.when((c == 0) & (s == 0))
    def _():
        pltpu.sync_copy(m0_hbm, m0_v)
        pltpu.sync_copy(m1_hbm, m1_v)

        @pl.loop(0, NQ // LANES)
        def _(i):
            idx_v[pl.ds(i * LANES, LANES)] = jnp.full((LANES,), -1, jnp.int32)

        iota = lax.iota(jnp.int32, LANES)
        dnums = lax.GatherDimensionNumbers(
            offset_dims=(), collapsed_slice_dims=(0,), start_index_map=(0,))

        @pl.loop(0, N_MCH)
        def _(t):
            m0c = m0_v[pl.ds(t * LANES, LANES)]
            m1c = m1_v[pl.ds(t * LANES, LANES)]
            dead = jnp.zeros((LANES,), jnp.bool_)
            for sh in range(1, LANES):
                idxs = jnp.minimum(iota + sh, LANES - 1)
                shifted = lax.gather(
                    m0c, idxs[:, None], dnums, (1,),
                    mode=lax.GatherScatterMode.PROMISE_IN_BOUNDS)
                dead = dead | ((shifted == m0c) & (iota + sh <= LANES - 1))
            keep = (~dead) & (m0c >= 0)
            plsc.store_scatter(idx_v, [m0c], m1c, mask=keep)

        pltpu.sync_copy(idx_v, idx_hbm)


@functools.partial(
    pl.kernel,
    out_type=jax.ShapeDtypeStruct((NC, NQ, H), jnp.float32),
    mesh=_mesh,
    scratch_types=[
        pltpu.VMEM((RCH,), jnp.int32),
        pltpu.VMEM((RCH,), jnp.int32),
        pltpu.VMEM((RCH, H), jnp.float32),
        pltpu.SemaphoreType.DMA,
    ],
)
def _gather_gx(gxflat_hbm, idx_hbm, out_hbm, idx_v, gidx_v, rows_v, sem):
    c = lax.axis_index("c")
    s = lax.axis_index("s")

    @pl.loop(s, N_RCH, step=NS)
    def _(t):
        pltpu.sync_copy(idx_hbm.at[pl.ds(t * RCH, RCH)], idx_v)

        @pl.loop(0, RCH // LANES)
        def _(i):
            w = idx_v[pl.ds(i * LANES, LANES)]
            gidx_v[pl.ds(i * LANES, LANES)] = jnp.maximum(w, 0) * 2 + c

        pltpu.async_copy(gxflat_hbm.at[gidx_v], rows_v, sem).wait()
        pltpu.sync_copy(rows_v, out_hbm.at[c].at[pl.ds(t * RCH, RCH)])


@functools.partial(
    pl.kernel,
    out_type=jax.ShapeDtypeStruct((NC, NQ, H), jnp.float32),
    mesh=_mesh,
    scratch_types=[
        pltpu.VMEM_SHARED((NQ, H), jnp.float32),
        pltpu.VMEM((ECH,), jnp.int32),
        pltpu.VMEM((ECH,), jnp.int32),
        pltpu.VMEM((ECH,), jnp.int32),
        pltpu.VMEM((ECH, H), jnp.float32),
        pltpu.VMEM((RCH, H), jnp.float32),
        pltpu.SemaphoreType.DMA,
    ],
)
def _seg_sum(qxflat_hbm, src_hbm, dst_hbm, agg_hbm,
             acc_sh, src_v, gidx_v, dst_v, rows_v, zero_v, sem):
    c = lax.axis_index("c")
    s = lax.axis_index("s")

    @pl.loop(0, RCH)
    def _(r):
        @pl.loop(0, H // LANES)
        def _(i):
            zero_v[r, pl.ds(i * LANES, LANES)] = jnp.zeros((LANES,), jnp.float32)

    @pl.loop(s, N_RCH, step=NS)
    def _(t):
        pltpu.sync_copy(zero_v, acc_sh.at[pl.ds(t * RCH, RCH)])

    plsc.subcore_barrier()

    @pl.loop(s, N_ECH, step=NS)
    def _(t):
        pltpu.sync_copy(src_hbm.at[pl.ds(t * ECH, ECH)], src_v)
        pltpu.sync_copy(dst_hbm.at[pl.ds(t * ECH, ECH)], dst_v)

        @pl.loop(0, ECH // LANES)
        def _(i):
            gidx_v[pl.ds(i * LANES, LANES)] = src_v[pl.ds(i * LANES, LANES)] * 2 + c

        pltpu.async_copy(qxflat_hbm.at[gidx_v], rows_v, sem).wait()
        pltpu.sync_copy(rows_v, acc_sh.at[dst_v], add=True)

    plsc.subcore_barrier()

    @pl.loop(s, N_RCH, step=NS)
    def _(t):
        pltpu.sync_copy(acc_sh.at[pl.ds(t * RCH, RCH)],
                        agg_hbm.at[c].at[pl.ds(t * RCH, RCH)])


def _mm_body(x_ref, w_ref, b_ref, o_ref):
    o_ref[...] = (
        jnp.dot(x_ref[...], w_ref[...], preferred_element_type=jnp.float32)
        + b_ref[...]
    )


def _input_linear(x, w, b):
    return pl.pallas_call(
        _mm_body,
        grid=(GRID,),
        in_specs=[
            pl.BlockSpec((RB, D), lambda i: (i, 0)),
            pl.BlockSpec((D, D), lambda i: (0, 0)),
            pl.BlockSpec((1, D), lambda i: (0, 0)),
        ],
        out_specs=pl.BlockSpec((RB, D), lambda i: (i, 0)),
        out_shape=jax.ShapeDtypeStruct((NQ, D), jnp.float32),
    )(x, w, b)


def _layer_body(qx_ref, a0_ref, a1_ref, w_ref, b_ref, o_ref):
    x = qx_ref[...] + jnp.concatenate([a0_ref[0], a1_ref[0]], axis=1)
    y = jnp.dot(x, w_ref[...], preferred_element_type=jnp.float32) + b_ref[...]
    o_ref[...] = jnp.maximum(y, 0.0)


def _gin_layer(qx, agg, w, b):
    return pl.pallas_call(
        _layer_body,
        grid=(GRID,),
        in_specs=[
            pl.BlockSpec((RB, D), lambda i: (i, 0)),
            pl.BlockSpec((1, RB, H), lambda i: (0, i, 0)),
            pl.BlockSpec((1, RB, H), lambda i: (1, i, 0)),
            pl.BlockSpec((D, D), lambda i: (0, 0)),
            pl.BlockSpec((1, D), lambda i: (0, 0)),
        ],
        out_specs=pl.BlockSpec((RB, D), lambda i: (i, 0)),
        out_shape=jax.ShapeDtypeStruct((NQ, D), jnp.float32),
    )(qx, agg, agg, w, b)


def _final_body(qx_ref, g0_ref, g1_ref, idx_ref, w1_ref, w2_ref, lb_ref,
                ow_ref, ob_ref, o_ref, acc_ref):
    i = pl.program_id(0)

    @pl.when(i == 0)
    def _():
        acc_ref[...] = jnp.full_like(acc_ref, -jnp.inf)

    mask = (idx_ref[...] >= 0).astype(jnp.float32)
    gx = jnp.concatenate([g0_ref[0], g1_ref[0]], axis=1) * mask
    x = (
        jnp.dot(qx_ref[...], w1_ref[...], preferred_element_type=jnp.float32)
        + jnp.dot(gx, w2_ref[...], preferred_element_type=jnp.float32)
        + lb_ref[...]
    )
    acc_ref[...] = jnp.maximum(acc_ref[...], x)

    @pl.when(i == pl.num_programs(0) - 1)
    def _():
        pooled = jnp.max(acc_ref[...], axis=0, keepdims=True)
        y = (
            jnp.dot(pooled, ow_ref[...], preferred_element_type=jnp.float32)
            + ob_ref[...]
        )
        o_ref[...] = jnp.maximum(y, 0.0)


def _final_stage(qx, gx, idx2d, w1, w2, lb, ow_pad, ob_pad):
    return pl.pallas_call(
        _final_body,
        grid=(GRID,),
        in_specs=[
            pl.BlockSpec((RB, D), lambda i: (i, 0)),
            pl.BlockSpec((1, RB, H), lambda i: (0, i, 0)),
            pl.BlockSpec((1, RB, H), lambda i: (1, i, 0)),
            pl.BlockSpec((RB, 1), lambda i: (i, 0)),
            pl.BlockSpec((D, D), lambda i: (0, 0)),
            pl.BlockSpec((D, D), lambda i: (0, 0)),
            pl.BlockSpec((1, D), lambda i: (0, 0)),
            pl.BlockSpec((D, H), lambda i: (0, 0)),
            pl.BlockSpec((1, H), lambda i: (0, 0)),
        ],
        out_specs=pl.BlockSpec((1, H), lambda i: (0, 0)),
        out_shape=jax.ShapeDtypeStruct((1, H), jnp.float32),
        scratch_shapes=[pltpu.VMEM((RB, D), jnp.float32)],
    )(qx, gx, gx, idx2d, w1, w2, lb, ow_pad, ob_pad)


def kernel(query_x, query_e, graph_x, matches, inq_W, inq_b, gin_W, gin_b,
           lin_W, lin_b, out_W, out_b):
    src = query_e[0]
    dst = query_e[1]

    mpad = jnp.pad(matches, ((0, 0), (0, MPAD - M)), constant_values=-1)
    idx = _build_idx(mpad[0], mpad[1])
    gx = _gather_gx(graph_x.reshape(NG * 2, H), idx)

    qx = _input_linear(query_x, inq_W, inq_b.reshape(1, D))
    for i in range(gin_W.shape[0]):
        agg = _seg_sum(qx.reshape(NQ * 2, H), src, dst)
        qx = _gin_layer(qx, agg, gin_W[i], gin_b[i].reshape(1, D))

    w1 = lin_W[:D]
    w2 = lin_W[D:]
    ow_pad = jnp.zeros((D, H), jnp.float32).at[:, :2].set(out_W)
    ob_pad = jnp.zeros((1, H), jnp.float32).at[:, :2].set(out_b.reshape(1, 2))

    out = _final_stage(qx, gx, idx.reshape(NQ, 1), w1, w2,
                       lin_b.reshape(1, D), ow_pad, ob_pad)
    return out[0, :2]


# speedup vs baseline: 6.4688x; 2.2379x over previous
import dataclasses
import functools
import jax
import jax.numpy as jnp
from jax import lax
from jax.experimental import pallas as pl
from jax.experimental.pallas import tpu as pltpu
from jax.experimental.pallas import tpu_sc as plsc

NQ = 10000
E = 160000
NG = 50000
D = 256
H = 128
M = 5000

NC = 2
NS = 16
LANES = 16

ECH = 128
N_ECH = E // ECH
RCH = 80
N_RCH = NQ // RCH

RB = 1000
GRID = NQ // RB

_mesh = plsc.VectorSubcoreMesh(core_axis_name="c", subcore_axis_name="s")

_no_layout_cp = pltpu.CompilerParams()
if "needs_layout_passes" in pltpu.CompilerParams.__dataclass_fields__:
    _no_layout_cp = dataclasses.replace(_no_layout_cp, needs_layout_passes=False)


MPAD = ((M + LANES - 1) // LANES) * LANES
N_MCH = MPAD // LANES


@functools.partial(
    pl.kernel,
    out_type=jax.ShapeDtypeStruct((NQ,), jnp.int32),
    mesh=_mesh,
    compiler_params=_no_layout_cp,
    scratch_types=[
        pltpu.VMEM((NQ,), jnp.int32),
        pltpu.VMEM((MPAD,), jnp.int32),
        pltpu.VMEM((MPAD,), jnp.int32),
    ],
)
def _build_idx(m0_hbm, m1_hbm, idx_hbm, idx_v, m0_v, m1_v):
    c = lax.axis_index("c")
    s = lax.axis_index("s")

    @pl.when((c == 0) & (s == 0))
    def _():
        pltpu.sync_copy(m0_hbm, m0_v)
        pltpu.sync_copy(m1_hbm, m1_v)

        @pl.loop(0, NQ // LANES)
        def _(i):
            idx_v[pl.ds(i * LANES, LANES)] = jnp.full((LANES,), -1, jnp.int32)

        iota = lax.iota(jnp.int32, LANES)
        dnums = lax.GatherDimensionNumbers(
            offset_dims=(), collapsed_slice_dims=(0,), start_index_map=(0,))

        @pl.loop(0, N_MCH)
        def _(t):
            m0c = m0_v[pl.ds(t * LANES, LANES)]
            m1c = m1_v[pl.ds(t * LANES, LANES)]
            dead = jnp.zeros((LANES,), jnp.bool_)
            for sh in range(1, LANES):
                idxs = jnp.minimum(iota + sh, LANES - 1)
                shifted = lax.gather(
                    m0c, idxs[:, None], dnums, (1,),
                    mode=lax.GatherScatterMode.PROMISE_IN_BOUNDS)
                dead = dead | ((shifted == m0c) & (iota + sh <= LANES - 1))
            keep = (~dead) & (m0c >= 0)
            plsc.store_scatter(idx_v, [m0c], m1c, mask=keep)

        pltpu.sync_copy(idx_v, idx_hbm)


GXB = 8
GX_FULL = N_RCH // NS * NS
GX_REST = N_RCH - GX_FULL


@functools.partial(
    pl.kernel,
    out_type=jax.ShapeDtypeStruct((NC, NQ, H), jnp.float32),
    mesh=_mesh,
    scratch_types=[
        pltpu.VMEM((NQ,), jnp.int32),
        pltpu.VMEM((GXB, RCH), jnp.int32),
        pltpu.VMEM((GXB, RCH, H), jnp.float32),
        pltpu.SemaphoreType.DMA,
        pltpu.SemaphoreType.DMA,
        pltpu.SemaphoreType.DMA,
    ],
)
def _gather_gx(gxflat_hbm, idx_hbm, out_hbm, idx_v, gidx_v, rows_v,
               lsem, gsem, osem):
    c = lax.axis_index("c")
    s = lax.axis_index("s")
    pltpu.async_copy(idx_hbm, idx_v, lsem).wait()

    def chunk_of(b):
        return 7 * s + b if b < 7 else GX_FULL + s

    def prep_issue(b):
        t = chunk_of(b)

        iota = lax.iota(jnp.int32, LANES)

        @pl.loop(0, RCH // LANES)
        def _(i):
            w = idx_v[pl.ds(t * RCH + i * LANES, LANES)]
            row = t * RCH + i * LANES + iota
            gidx_v[b, pl.ds(i * LANES, LANES)] = (
                jnp.where(w >= 0, w, row) * 2 + c)

        pltpu.async_copy(gxflat_hbm.at[gidx_v.at[b]], rows_v.at[b], gsem)

    def wait_issue_out(b):
        t = chunk_of(b)
        pltpu.make_async_copy(
            gxflat_hbm.at[gidx_v.at[b]], rows_v.at[b], gsem).wait()
        pltpu.async_copy(rows_v.at[b], out_hbm.at[c].at[pl.ds(t * RCH, RCH)],
                         osem)

    def drain_out(b):
        t = chunk_of(b)
        pltpu.make_async_copy(
            rows_v.at[b], out_hbm.at[c].at[pl.ds(t * RCH, RCH)], osem).wait()

    for b in range(7):
        prep_issue(b)

    @pl.when(s < GX_REST)
    def _():
        prep_issue(7)

    for b in range(7):
        wait_issue_out(b)

    @pl.when(s < GX_REST)
    def _():
        wait_issue_out(7)

    for b in range(7):
        drain_out(b)

    @pl.when(s < GX_REST)
    def _():
        drain_out(7)


EPS = E // NS
SCH = 80
SSB = 4
N_LCH = EPS // SCH
N_SUP = N_LCH // SSB
N_TAIL = N_LCH - N_SUP * SSB


@functools.partial(
    pl.kernel,
    out_type=jax.ShapeDtypeStruct((NC, NQ, H), jnp.float32),
    mesh=_mesh,
    scratch_types=[
        pltpu.VMEM_SHARED((NQ, H), jnp.float32),
        pltpu.VMEM((SSB, SCH), jnp.int32),
        pltpu.VMEM((SSB, SCH), jnp.int32),
        pltpu.VMEM((SSB, SCH), jnp.int32),
        pltpu.VMEM((SSB, SCH, H), jnp.float32),
        pltpu.SemaphoreType.DMA,
        pltpu.SemaphoreType.DMA,
        pltpu.SemaphoreType.DMA,
        pltpu.SemaphoreType.DMA,
    ],
)
def _seg_sum(qxflat_hbm, src_hbm, dst_hbm, agg_hbm,
             acc_sh, sstg_v, gidx_v, dstc_v, rows_v,
             isem, zsem, gsem, ssem):
    c = lax.axis_index("c")
    s = lax.axis_index("s")
    base = s * EPS

    def idx_issue(tl, b):
        goff = base + tl * SCH
        pltpu.async_copy(src_hbm.at[pl.ds(goff, SCH)], sstg_v.at[b], isem)
        pltpu.async_copy(dst_hbm.at[pl.ds(goff, SCH)], dstc_v.at[b], isem)

    def idx_wait(tl, b):
        goff = base + tl * SCH
        pltpu.make_async_copy(
            src_hbm.at[pl.ds(goff, SCH)], sstg_v.at[b], isem).wait()
        pltpu.make_async_copy(
            dst_hbm.at[pl.ds(goff, SCH)], dstc_v.at[b], isem).wait()

    def gather_issue(b):
        @pl.loop(0, SCH // LANES)
        def _(i):
            sv = sstg_v[b, pl.ds(i * LANES, LANES)]
            gidx_v[b, pl.ds(i * LANES, LANES)] = sv * 2 + c

        pltpu.async_copy(qxflat_hbm.at[gidx_v.at[b]], rows_v.at[b], gsem)

    def gather_wait(b):
        pltpu.make_async_copy(
            qxflat_hbm.at[gidx_v.at[b]], rows_v.at[b], gsem).wait()

    def scatter_issue(b):
        pltpu.async_copy(rows_v.at[b], acc_sh.at[dstc_v.at[b]], ssem,
                         add=True)

    def scatter_wait(b):
        pltpu.make_async_copy(
            rows_v.at[b], acc_sh.at[dstc_v.at[b]], ssem).wait()

    for b in range(SSB):
        idx_issue(b, b)

    @pl.loop(0, SCH)
    def _(r):
        @pl.loop(0, H // LANES)
        def _(i):
            rows_v[0, r, pl.ds(i * LANES, LANES)] = (
                jnp.zeros((LANES,), jnp.float32))

    @pl.loop(s, N_RCH, step=NS)
    def _(t):
        pltpu.async_copy(rows_v.at[0], acc_sh.at[pl.ds(t * RCH, RCH)], zsem)

    @pl.loop(s, N_RCH, step=NS)
    def _(t):
        pltpu.make_async_copy(rows_v.at[0], acc_sh.at[pl.ds(t * RCH, RCH)],
                              zsem).wait()

    plsc.subcore_barrier()

    @pl.loop(0, N_SUP)
    def _(m):
        for b in range(SSB):
            idx_wait(m * SSB + b, b)
            gather_issue(b)
        for b in range(SSB):
            gather_wait(b)
            scatter_issue(b)

        for b in range(SSB):
            scatter_wait(b)

            @pl.when(m < N_SUP - 1)
            def _():
                idx_issue((m + 1) * SSB + b, b)

            if b < N_TAIL:
                @pl.when(m == N_SUP - 1)
                def _():
                    idx_issue(N_SUP * SSB + b, b)

    for b in range(N_TAIL):
        idx_wait(N_SUP * SSB + b, b)
        gather_issue(b)
    for b in range(N_TAIL):
        gather_wait(b)
        scatter_issue(b)
    for b in range(N_TAIL):
        scatter_wait(b)

    plsc.subcore_barrier()

    @pl.loop(s, N_RCH, step=NS)
    def _(t):
        pltpu.async_copy(acc_sh.at[pl.ds(t * RCH, RCH)],
                         agg_hbm.at[c].at[pl.ds(t * RCH, RCH)], zsem)

    @pl.loop(s, N_RCH, step=NS)
    def _(t):
        pltpu.make_async_copy(acc_sh.at[pl.ds(t * RCH, RCH)],
                              agg_hbm.at[c].at[pl.ds(t * RCH, RCH)],
                              zsem).wait()


def _mm_body(x_ref, w_ref, b_ref, o_ref):
    o_ref[...] = (
        jnp.dot(x_ref[...], w_ref[...], preferred_element_type=jnp.float32)
        + b_ref[...]
    )


def _input_linear(x, w, b):
    return pl.pallas_call(
        _mm_body,
        grid=(GRID,),
        in_specs=[
            pl.BlockSpec((RB, D), lambda i: (i, 0)),
            pl.BlockSpec((D, D), lambda i: (0, 0)),
            pl.BlockSpec((1, D), lambda i: (0, 0)),
        ],
        out_specs=pl.BlockSpec((RB, D), lambda i: (i, 0)),
        out_shape=jax.ShapeDtypeStruct((NQ, D), jnp.float32),
    )(x, w, b)


def _layer_body(qx_ref, a0_ref, a1_ref, w_ref, b_ref, o_ref):
    x = qx_ref[...] + jnp.concatenate([a0_ref[0], a1_ref[0]], axis=1)
    y = jnp.dot(x, w_ref[...], preferred_element_type=jnp.float32) + b_ref[...]
    o_ref[...] = jnp.maximum(y, 0.0)


def _gin_layer(qx, agg, w, b):
    return pl.pallas_call(
        _layer_body,
        grid=(GRID,),
        in_specs=[
            pl.BlockSpec((RB, D), lambda i: (i, 0)),
            pl.BlockSpec((1, RB, H), lambda i: (0, i, 0)),
            pl.BlockSpec((1, RB, H), lambda i: (1, i, 0)),
            pl.BlockSpec((D, D), lambda i: (0, 0)),
            pl.BlockSpec((1, D), lambda i: (0, 0)),
        ],
        out_specs=pl.BlockSpec((RB, D), lambda i: (i, 0)),
        out_shape=jax.ShapeDtypeStruct((NQ, D), jnp.float32),
    )(qx, agg, agg, w, b)


def _final_body(qx_ref, g0_ref, g1_ref, idx_ref, w1_ref, w2_ref, lb_ref,
                ow_ref, ob_ref, o_ref, acc_ref):
    i = pl.program_id(0)

    @pl.when(i == 0)
    def _():
        acc_ref[...] = jnp.full_like(acc_ref, -jnp.inf)

    mask = (idx_ref[...] >= 0).astype(jnp.float32)
    gx = jnp.concatenate([g0_ref[0], g1_ref[0]], axis=1) * mask
    x = (
        jnp.dot(qx_ref[...], w1_ref[...], preferred_element_type=jnp.float32)
        + jnp.dot(gx, w2_ref[...], preferred_element_type=jnp.float32)
        + lb_ref[...]
    )
    acc_ref[...] = jnp.maximum(acc_ref[...], x)

    @pl.when(i == pl.num_programs(0) - 1)
    def _():
        pooled = jnp.max(acc_ref[...], axis=0, keepdims=True)
        y = (
            jnp.dot(pooled, ow_ref[...], preferred_element_type=jnp.float32)
            + ob_ref[...]
        )
        o_ref[...] = jnp.maximum(y, 0.0)


def _final_stage(qx, gx, idx2d, w1, w2, lb, ow_pad, ob_pad):
    return pl.pallas_call(
        _final_body,
        grid=(GRID,),
        in_specs=[
            pl.BlockSpec((RB, D), lambda i: (i, 0)),
            pl.BlockSpec((1, RB, H), lambda i: (0, i, 0)),
            pl.BlockSpec((1, RB, H), lambda i: (1, i, 0)),
            pl.BlockSpec((RB, 1), lambda i: (i, 0)),
            pl.BlockSpec((D, D), lambda i: (0, 0)),
            pl.BlockSpec((D, D), lambda i: (0, 0)),
            pl.BlockSpec((1, D), lambda i: (0, 0)),
            pl.BlockSpec((D, H), lambda i: (0, 0)),
            pl.BlockSpec((1, H), lambda i: (0, 0)),
        ],
        out_specs=pl.BlockSpec((1, H), lambda i: (0, 0)),
        out_shape=jax.ShapeDtypeStruct((1, H), jnp.float32),
        scratch_shapes=[pltpu.VMEM((RB, D), jnp.float32)],
    )(qx, gx, gx, idx2d, w1, w2, lb, ow_pad, ob_pad)


def kernel(query_x, query_e, graph_x, matches, inq_W, inq_b, gin_W, gin_b,
           lin_W, lin_b, out_W, out_b):
    src = query_e[0]
    dst = query_e[1]

    mpad = jnp.pad(matches, ((0, 0), (0, MPAD - M)), constant_values=-1)
    idx = _build_idx(mpad[0], mpad[1])
    gx = _gather_gx(graph_x.reshape(NG * 2, H), idx)

    qx = _input_linear(query_x, inq_W, inq_b.reshape(1, D))
    for i in range(gin_W.shape[0]):
        agg = _seg_sum(qx.reshape(NQ * 2, H), src, dst)
        qx = _gin_layer(qx, agg, gin_W[i], gin_b[i].reshape(1, D))

    w1 = lin_W[:D]
    w2 = lin_W[D:]
    ow_pad = jnp.zeros((D, H), jnp.float32).at[:, :2].set(out_W)
    ob_pad = jnp.zeros((1, H), jnp.float32).at[:, :2].set(out_b.reshape(1, 2))

    out = _final_stage(qx, gx, idx.reshape(NQ, 1), w1, w2,
                       lin_b.reshape(1, D), ow_pad, ob_pad)
    return out[0, :2]
